# baseline (speedup 1.0000x reference)
import functools

import jax
import jax.numpy as jnp
from jax import lax
from jax.experimental import pallas as pl
from jax.experimental.pallas import tpu as pltpu
from jax.experimental.pallas import tpu_sc as plsc

NC = 2
NS = 16
CHUNK = 128
KB = 8


def _round_up(a, b):
    return (a + b - 1) // b * b


def _sc_aggregate(data, colh, rowh, zeros2d, zeros1d, ones1, n_pad,
                  nch0, nch1, with_deg):
    D = data.shape[1]
    rpt = n_pad // NS
    mesh = plsc.VectorSubcoreMesh(core_axis_name="c", subcore_axis_name="s")

    out_type = [jax.ShapeDtypeStruct((NC, n_pad, D), jnp.float32)]
    deg_scratch = []
    if with_deg:
        out_type.append(jax.ShapeDtypeStruct((NC, n_pad), jnp.float32))
        deg_scratch = [pltpu.VMEM_SHARED((n_pad,), jnp.float32)]

    @functools.partial(
        pl.kernel,
        out_type=tuple(out_type),
        mesh=mesh,
        scratch_types=[
            pltpu.VMEM((2, KB, CHUNK), jnp.int32),
            pltpu.VMEM((2, KB, CHUNK), jnp.int32),
            pltpu.VMEM((2, CHUNK, D), jnp.float32),
            pltpu.VMEM((CHUNK,), jnp.float32),
            pltpu.VMEM_SHARED((n_pad, D), jnp.float32),
            *deg_scratch,
            pltpu.SemaphoreType.DMA,
            pltpu.SemaphoreType.DMA,
        ],
    )
    def k(data_hbm, col_hbm, row_hbm, z2_hbm, z1_hbm, ones_hbm, *rest):
        if with_deg:
            (agg_out, deg_out, col_w, row_w, gbuf, ones_v, sh_agg, sh_deg,
             gsem, isem) = rest
        else:
            agg_out, col_w, row_w, gbuf, ones_v, sh_agg, gsem, isem = rest
        c = lax.axis_index("c")
        s = lax.axis_index("s")
        rbase = s * rpt
        with jax.named_scope("zero"):
            pltpu.sync_copy(z2_hbm, sh_agg.at[pl.ds(rbase, rpt)])
            if with_deg:
                pltpu.sync_copy(z1_hbm, sh_deg.at[pl.ds(rbase, rpt)])
                pltpu.sync_copy(ones_hbm, ones_v)
            plsc.subcore_barrier()

        def run(base, nch):
            nw = nch // KB
            pltpu.async_copy(col_hbm.at[pl.ds(base, KB)], col_w.at[0], isem)
            pltpu.async_copy(row_hbm.at[pl.ds(base, KB)], row_w.at[0], isem)

            @pl.loop(0, nw, step=2)
            def _(ww):
                for slot in range(2):
                    w = ww + slot
                    wb = base + w * KB
                    pltpu.make_async_copy(
                        col_hbm.at[pl.ds(wb, KB)], col_w.at[slot], isem).wait()
                    pltpu.make_async_copy(
                        row_hbm.at[pl.ds(wb, KB)], row_w.at[slot], isem).wait()

                    @pl.when(w + 1 < nw)
                    def _():
                        nb = base + (w + 1) * KB
                        pltpu.async_copy(col_hbm.at[pl.ds(nb, KB)],
                                         col_w.at[1 - slot], isem)
                        pltpu.async_copy(row_hbm.at[pl.ds(nb, KB)],
                                         row_w.at[1 - slot], isem)

                    @pl.loop(0, KB, step=2)
                    def _(jj):
                        da = pltpu.async_copy(
                            data_hbm.at[col_w.at[slot, jj]], gbuf.at[0], gsem)
                        db = pltpu.async_copy(
                            data_hbm.at[col_w.at[slot, jj + 1]], gbuf.at[1],
                            gsem)
                        da.wait()
                        pltpu.sync_copy(gbuf.at[0],
                                        sh_agg.at[row_w.at[slot, jj]],
                                        add=True)
                        if with_deg:
                            pltpu.sync_copy(ones_v,
                                            sh_deg.at[row_w.at[slot, jj]],
                                            add=True)
                        db.wait()
                        pltpu.sync_copy(gbuf.at[1],
                                        sh_agg.at[row_w.at[slot, jj + 1]],
                                        add=True)
                        if with_deg:
                            pltpu.sync_copy(ones_v,
                                            sh_deg.at[row_w.at[slot, jj + 1]],
                                            add=True)

        with jax.named_scope("edges"):
            @pl.when(c == 0)
            def _():
                run(s * nch0, nch0)

            if nch1 > 0:
                @pl.when(c == 1)
                def _():
                    run(NS * nch0 + s * nch1, nch1)

            plsc.subcore_barrier()

        with jax.named_scope("flush"):
            pltpu.sync_copy(sh_agg.at[pl.ds(rbase, rpt)],
                            agg_out.at[c, pl.ds(rbase, rpt)])
            if with_deg:
                pltpu.sync_copy(sh_deg.at[pl.ds(rbase, rpt)],
                                deg_out.at[c, pl.ds(rbase, rpt)])

    return k(data, colh, rowh, zeros2d, zeros1d, ones1)


def _tc_linear(parts, degs3, W, b, relu):
    n_pad, D = parts.shape[1], parts.shape[2]
    H = W.shape[1]
    BLK = 512

    def body(p_ref, d_ref, w_ref, b_ref, o_ref):
        agg = p_ref[0] + p_ref[1]
        deg = d_ref[0] + d_ref[1]
        dinv = jnp.where(deg > 0, lax.rsqrt(jnp.maximum(deg, 1.0)), 0.0)
        out = jnp.dot(agg * dinv, w_ref[...],
                      preferred_element_type=jnp.float32,
                      precision=lax.Precision.HIGHEST)
        out = out + b_ref[...]
        if relu:
            out = jnp.maximum(out, 0.0)
        o_ref[...] = out

    return pl.pallas_call(
        body,
        grid=(n_pad // BLK,),
        in_specs=[
            pl.BlockSpec((NC, BLK, D), lambda i: (0, i, 0)),
            pl.BlockSpec((NC, BLK, 1), lambda i: (0, i, 0)),
            pl.BlockSpec((D, H), lambda i: (0, 0)),
            pl.BlockSpec((1, H), lambda i: (0, 0)),
        ],
        out_specs=pl.BlockSpec((BLK, H), lambda i: (i, 0)),
        out_shape=jax.ShapeDtypeStruct((n_pad, H), jnp.float32),
    )(parts, degs3, W, b.reshape(1, H))


def kernel(x, edge_index, W1, b1, W2, b2, size):
    N, D = x.shape
    E = edge_index.shape[1]
    H = W1.shape[1]

    tot = _round_up(E, NS * CHUNK * 4 * KB) // (NS * CHUNK)
    nch1 = 0
    nch0 = tot - nch1
    e_pad = NS * tot * CHUNK
    n_pad = _round_up(N + 1, 1024)

    row = edge_index[0]
    col = edge_index[1]
    pad = e_pad - E
    dummy = N + jnp.arange(pad, dtype=jnp.int32) % (n_pad - N)
    rowp = jnp.concatenate([row, dummy])
    colp = jnp.concatenate([col, jnp.zeros((pad,), jnp.int32)])
    rowh = rowp.reshape(NS * tot, CHUNK)
    colh = colp.reshape(NS * tot, CHUNK)

    rpt = n_pad // NS
    zeros2d = jnp.zeros((rpt, D), jnp.float32)
    zeros1d = jnp.zeros((rpt,), jnp.float32)
    ones1 = jnp.ones((CHUNK,), jnp.float32)

    agg1, deg = _sc_aggregate(x, colh, rowh, zeros2d, zeros1d, ones1,
                              n_pad, nch0, nch1, with_deg=True)
    degs3 = deg.reshape(NC, n_pad, 1)
    h = _tc_linear(agg1, degs3, W1, b1, relu=True)
    (agg2,) = _sc_aggregate(h, colh, rowh, zeros2d, zeros1d, ones1,
                            n_pad, nch0, nch1, with_deg=False)
    logits = _tc_linear(agg2, degs3, W2, b2, relu=False)
    return logits[:N]


# speedup vs baseline: 4.1110x; 4.1110x over previous
import functools

import jax
import jax.numpy as jnp
from jax import lax
from jax.experimental import pallas as pl
from jax.experimental.pallas import tpu as pltpu
from jax.experimental.pallas import tpu_sc as plsc

NC = 2
NS = 16
CHUNK = 128
KB = 8


def _round_up(a, b):
    return (a + b - 1) // b * b


def _sc_aggregate(data, colh, rowh, zeros2d, zeros1d, ones1, n_pad,
                  nch0, nch1, with_deg):
    D = data.shape[1]
    rpt = n_pad // NS
    mesh = plsc.VectorSubcoreMesh(core_axis_name="c", subcore_axis_name="s")

    out_type = [jax.ShapeDtypeStruct((NC, n_pad, D), jnp.float32)]
    deg_scratch = []
    if with_deg:
        out_type.append(jax.ShapeDtypeStruct((NC, n_pad), jnp.float32))
        deg_scratch = [pltpu.VMEM_SHARED((n_pad,), jnp.float32)]

    @functools.partial(
        pl.kernel,
        out_type=tuple(out_type),
        mesh=mesh,
        scratch_types=[
            pltpu.VMEM((2, KB, CHUNK), jnp.int32),
            pltpu.VMEM((2, KB, CHUNK), jnp.int32),
            pltpu.VMEM((2, CHUNK, D), jnp.float32),
            pltpu.VMEM((CHUNK,), jnp.float32),
            pltpu.VMEM_SHARED((n_pad, D), jnp.float32),
            *deg_scratch,
            pltpu.SemaphoreType.DMA,
            pltpu.SemaphoreType.DMA,
        ],
    )
    def k(data_hbm, col_hbm, row_hbm, z2_hbm, z1_hbm, ones_hbm, *rest):
        if with_deg:
            (agg_out, deg_out, col_w, row_w, gbuf, ones_v, sh_agg, sh_deg,
             gsem, isem) = rest
        else:
            agg_out, col_w, row_w, gbuf, ones_v, sh_agg, gsem, isem = rest
        c = lax.axis_index("c")
        s = lax.axis_index("s")
        rbase = s * rpt
        with jax.named_scope("zero"):
            pltpu.sync_copy(z2_hbm, sh_agg.at[pl.ds(rbase, rpt)])
            if with_deg:
                pltpu.sync_copy(z1_hbm, sh_deg.at[pl.ds(rbase, rpt)])
                pltpu.sync_copy(ones_hbm, ones_v)
            plsc.subcore_barrier()

        def run(base, nch):
            nw = nch // KB
            pltpu.async_copy(col_hbm.at[pl.ds(base, KB)], col_w.at[0], isem)
            pltpu.async_copy(row_hbm.at[pl.ds(base, KB)], row_w.at[0], isem)

            @pl.loop(0, nw, step=2)
            def _(ww):
                for slot in range(2):
                    w = ww + slot
                    wb = base + w * KB
                    pltpu.make_async_copy(
                        col_hbm.at[pl.ds(wb, KB)], col_w.at[slot], isem).wait()
                    pltpu.make_async_copy(
                        row_hbm.at[pl.ds(wb, KB)], row_w.at[slot], isem).wait()

                    @pl.when(w + 1 < nw)
                    def _():
                        nb = base + (w + 1) * KB
                        pltpu.async_copy(col_hbm.at[pl.ds(nb, KB)],
                                         col_w.at[1 - slot], isem)
                        pltpu.async_copy(row_hbm.at[pl.ds(nb, KB)],
                                         row_w.at[1 - slot], isem)

                    @pl.loop(0, KB, step=2)
                    def _(jj):
                        da = pltpu.async_copy(
                            data_hbm.at[col_w.at[slot, jj]], gbuf.at[0], gsem)
                        db = pltpu.async_copy(
                            data_hbm.at[col_w.at[slot, jj + 1]], gbuf.at[1],
                            gsem)
                        da.wait()
                        pltpu.sync_copy(gbuf.at[0],
                                        sh_agg.at[row_w.at[slot, jj]],
                                        add=True)
                        if with_deg:
                            pltpu.sync_copy(ones_v,
                                            sh_deg.at[row_w.at[slot, jj]],
                                            add=True)
                        db.wait()
                        pltpu.sync_copy(gbuf.at[1],
                                        sh_agg.at[row_w.at[slot, jj + 1]],
                                        add=True)
                        if with_deg:
                            pltpu.sync_copy(ones_v,
                                            sh_deg.at[row_w.at[slot, jj + 1]],
                                            add=True)

        with jax.named_scope("edges"):
            @pl.when(c == 0)
            def _():
                run(s * nch0, nch0)

            if nch1 > 0:
                @pl.when(c == 1)
                def _():
                    run(NS * nch0 + s * nch1, nch1)

            plsc.subcore_barrier()

        with jax.named_scope("flush"):
            pltpu.sync_copy(sh_agg.at[pl.ds(rbase, rpt)],
                            agg_out.at[c, pl.ds(rbase, rpt)])
            if with_deg:
                pltpu.sync_copy(sh_deg.at[pl.ds(rbase, rpt)],
                                deg_out.at[c, pl.ds(rbase, rpt)])

    return k(data, colh, rowh, zeros2d, zeros1d, ones1)


def _tc_linear(parts, degs3, W, b, relu):
    n_pad, D = parts.shape[1], parts.shape[2]
    H = W.shape[1]
    BLK = 512

    def body(p_ref, d_ref, w_ref, b_ref, o_ref):
        agg = p_ref[0] + p_ref[1]
        deg = d_ref[0] + d_ref[1]
        dinv = jnp.where(deg > 0, lax.rsqrt(jnp.maximum(deg, 1.0)), 0.0)
        out = jnp.dot(agg * dinv, w_ref[...],
                      preferred_element_type=jnp.float32,
                      precision=lax.Precision.HIGHEST)
        out = out + b_ref[...]
        if relu:
            out = jnp.maximum(out, 0.0)
        o_ref[...] = out

    return pl.pallas_call(
        body,
        grid=(n_pad // BLK,),
        in_specs=[
            pl.BlockSpec((NC, BLK, D), lambda i: (0, i, 0)),
            pl.BlockSpec((NC, BLK, 1), lambda i: (0, i, 0)),
            pl.BlockSpec((D, H), lambda i: (0, 0)),
            pl.BlockSpec((1, H), lambda i: (0, 0)),
        ],
        out_specs=pl.BlockSpec((BLK, H), lambda i: (i, 0)),
        out_shape=jax.ShapeDtypeStruct((n_pad, H), jnp.float32),
    )(parts, degs3, W, b.reshape(1, H))


def kernel(x, edge_index, W1, b1, W2, b2, size):
    N, D = x.shape
    E = edge_index.shape[1]
    H = W1.shape[1]

    tot = _round_up(E, NS * CHUNK * 4 * KB) // (NS * CHUNK)
    nch1 = tot // 2
    nch0 = tot - nch1
    e_pad = NS * tot * CHUNK
    n_pad = _round_up(N + 1, 1024)

    row = edge_index[0]
    col = edge_index[1]
    pad = e_pad - E
    arange_pad = jnp.arange(pad, dtype=jnp.int32)
    rowp = jnp.concatenate([row, N + arange_pad % (n_pad - N)])
    colp = jnp.concatenate([col, arange_pad % N])
    rowh = rowp.reshape(NS * tot, CHUNK)
    colh = colp.reshape(NS * tot, CHUNK)

    rpt = n_pad // NS
    zeros2d = jnp.zeros((rpt, D), jnp.float32)
    zeros1d = jnp.zeros((rpt,), jnp.float32)
    ones1 = jnp.ones((CHUNK,), jnp.float32)

    agg1, deg = _sc_aggregate(x, colh, rowh, zeros2d, zeros1d, ones1,
                              n_pad, nch0, nch1, with_deg=True)
    degs3 = deg.reshape(NC, n_pad, 1)
    h = _tc_linear(agg1, degs3, W1, b1, relu=True)
    (agg2,) = _sc_aggregate(h, colh, rowh, zeros2d, zeros1d, ones1,
                            n_pad, nch0, nch1, with_deg=False)
    logits = _tc_linear(agg2, degs3, W2, b2, relu=False)
    return logits[:N]


# speedup vs baseline: 4.5099x; 1.0970x over previous
import functools

import jax
import jax.numpy as jnp
from jax import lax
from jax.experimental import pallas as pl
from jax.experimental.pallas import tpu as pltpu
from jax.experimental.pallas import tpu_sc as plsc

NC = 2
NS = 16
CHUNK = 128
KB = 8


def _round_up(a, b):
    return (a + b - 1) // b * b


def _sc_aggregate(data, colh, rowh, zeros2d, zeros1d, ones1, n_pad,
                  nch0, nch1, with_deg):
    D = data.shape[1]
    rpt = n_pad // NS
    mesh = plsc.VectorSubcoreMesh(core_axis_name="c", subcore_axis_name="s")

    out_type = [jax.ShapeDtypeStruct((NC, n_pad, D), jnp.float32)]
    deg_scratch = []
    if with_deg:
        out_type.append(jax.ShapeDtypeStruct((NC, n_pad), jnp.float32))
        deg_scratch = [pltpu.VMEM_SHARED((n_pad,), jnp.float32)]

    @functools.partial(
        pl.kernel,
        out_type=tuple(out_type),
        mesh=mesh,
        scratch_types=[
            pltpu.VMEM((2, KB, CHUNK), jnp.int32),
            pltpu.VMEM((2, KB, CHUNK), jnp.int32),
            pltpu.VMEM((2, CHUNK, D), jnp.float32),
            pltpu.VMEM((CHUNK,), jnp.float32),
            pltpu.VMEM_SHARED((n_pad, D), jnp.float32),
            *deg_scratch,
            pltpu.SemaphoreType.DMA,
            pltpu.SemaphoreType.DMA,
            pltpu.SemaphoreType.DMA,
        ],
    )
    def k(data_hbm, col_hbm, row_hbm, z2_hbm, z1_hbm, ones_hbm, *rest):
        if with_deg:
            (agg_out, deg_out, col_w, row_w, gbuf, ones_v, sh_agg, sh_deg,
             gsem, isem, ssem) = rest
        else:
            (agg_out, col_w, row_w, gbuf, ones_v, sh_agg,
             gsem, isem, ssem) = rest
        c = lax.axis_index("c")
        s = lax.axis_index("s")
        rbase = s * rpt
        with jax.named_scope("zero"):
            pltpu.sync_copy(z2_hbm, sh_agg.at[pl.ds(rbase, rpt)])
            if with_deg:
                pltpu.sync_copy(z1_hbm, sh_deg.at[pl.ds(rbase, rpt)])
                pltpu.sync_copy(ones_hbm, ones_v)
            plsc.subcore_barrier()

        def run(base, nch):
            nw = nch // KB

            def gath_start(slot, jj, b):
                pltpu.async_copy(data_hbm.at[col_w.at[slot, jj]], gbuf.at[b],
                                 gsem)

            def gath_wait(slot, jj, b):
                pltpu.make_async_copy(data_hbm.at[col_w.at[slot, jj]],
                                      gbuf.at[b], gsem).wait()

            def scat_start(slot, jj, b):
                pltpu.async_copy(gbuf.at[b], sh_agg.at[row_w.at[slot, jj]],
                                 ssem, add=True)

            def scat_wait(slot, jj, b):
                pltpu.make_async_copy(gbuf.at[b],
                                      sh_agg.at[row_w.at[slot, jj]],
                                      ssem).wait()

            pltpu.async_copy(col_hbm.at[pl.ds(base, KB)], col_w.at[0], isem)
            pltpu.async_copy(row_hbm.at[pl.ds(base, KB)], row_w.at[0], isem)

            @pl.loop(0, nw, step=2)
            def _(ww):
                for slot in range(2):
                    w = ww + slot
                    wb = base + w * KB
                    pltpu.make_async_copy(
                        col_hbm.at[pl.ds(wb, KB)], col_w.at[slot], isem).wait()
                    pltpu.make_async_copy(
                        row_hbm.at[pl.ds(wb, KB)], row_w.at[slot], isem).wait()

                    @pl.when(w + 1 < nw)
                    def _():
                        nb = base + (w + 1) * KB
                        pltpu.async_copy(col_hbm.at[pl.ds(nb, KB)],
                                         col_w.at[1 - slot], isem)
                        pltpu.async_copy(row_hbm.at[pl.ds(nb, KB)],
                                         row_w.at[1 - slot], isem)

                    gath_start(slot, 0, 0)
                    gath_start(slot, 1, 1)

                    @pl.loop(0, KB, step=2)
                    def _(jj):
                        gath_wait(slot, jj, 0)
                        scat_start(slot, jj, 0)
                        if with_deg:
                            pltpu.sync_copy(ones_v,
                                            sh_deg.at[row_w.at[slot, jj]],
                                            add=True)
                        gath_wait(slot, jj + 1, 1)
                        scat_start(slot, jj + 1, 1)
                        if with_deg:
                            pltpu.sync_copy(ones_v,
                                            sh_deg.at[row_w.at[slot, jj + 1]],
                                            add=True)

                        @pl.when(jj + 2 < KB)
                        def _():
                            scat_wait(slot, jj, 0)
                            gath_start(slot, jj + 2, 0)
                            scat_wait(slot, jj + 1, 1)
                            gath_start(slot, jj + 3, 1)

                    scat_wait(slot, KB - 2, 0)
                    scat_wait(slot, KB - 1, 1)

        with jax.named_scope("edges"):
            @pl.when(c == 0)
            def _():
                run(s * nch0, nch0)

            if nch1 > 0:
                @pl.when(c == 1)
                def _():
                    run(NS * nch0 + s * nch1, nch1)

            plsc.subcore_barrier()

        with jax.named_scope("flush"):
            pltpu.sync_copy(sh_agg.at[pl.ds(rbase, rpt)],
                            agg_out.at[c, pl.ds(rbase, rpt)])
            if with_deg:
                pltpu.sync_copy(sh_deg.at[pl.ds(rbase, rpt)],
                                deg_out.at[c, pl.ds(rbase, rpt)])

    return k(data, colh, rowh, zeros2d, zeros1d, ones1)


def _tc_linear(parts, degs3, W, b, relu):
    n_pad, D = parts.shape[1], parts.shape[2]
    H = W.shape[1]
    BLK = 512

    def body(p_ref, d_ref, w_ref, b_ref, o_ref):
        agg = p_ref[0] + p_ref[1]
        deg = d_ref[0] + d_ref[1]
        dinv = jnp.where(deg > 0, lax.rsqrt(jnp.maximum(deg, 1.0)), 0.0)
        out = jnp.dot(agg * dinv, w_ref[...],
                      preferred_element_type=jnp.float32,
                      precision=lax.Precision.HIGHEST)
        out = out + b_ref[...]
        if relu:
            out = jnp.maximum(out, 0.0)
        o_ref[...] = out

    return pl.pallas_call(
        body,
        grid=(n_pad // BLK,),
        in_specs=[
            pl.BlockSpec((NC, BLK, D), lambda i: (0, i, 0)),
            pl.BlockSpec((NC, BLK, 1), lambda i: (0, i, 0)),
            pl.BlockSpec((D, H), lambda i: (0, 0)),
            pl.BlockSpec((1, H), lambda i: (0, 0)),
        ],
        out_specs=pl.BlockSpec((BLK, H), lambda i: (i, 0)),
        out_shape=jax.ShapeDtypeStruct((n_pad, H), jnp.float32),
    )(parts, degs3, W, b.reshape(1, H))


def kernel(x, edge_index, W1, b1, W2, b2, size):
    N, D = x.shape
    E = edge_index.shape[1]
    H = W1.shape[1]

    tot = _round_up(E, NS * CHUNK * 4 * KB) // (NS * CHUNK)
    nch1 = tot // 2
    nch0 = tot - nch1
    e_pad = NS * tot * CHUNK
    n_pad = _round_up(N + 1, 1024)

    row = edge_index[0]
    col = edge_index[1]
    pad = e_pad - E
    arange_pad = jnp.arange(pad, dtype=jnp.int32)
    rowp = jnp.concatenate([row, N + arange_pad % (n_pad - N)])
    colp = jnp.concatenate([col, arange_pad % N])
    rowh = rowp.reshape(NS * tot, CHUNK)
    colh = colp.reshape(NS * tot, CHUNK)

    rpt = n_pad // NS
    zeros2d = jnp.zeros((rpt, D), jnp.float32)
    zeros1d = jnp.zeros((rpt,), jnp.float32)
    ones1 = jnp.ones((CHUNK,), jnp.float32)

    agg1, deg = _sc_aggregate(x, colh, rowh, zeros2d, zeros1d, ones1,
                              n_pad, nch0, nch1, with_deg=True)
    degs3 = deg.reshape(NC, n_pad, 1)
    h = _tc_linear(agg1, degs3, W1, b1, relu=True)
    (agg2,) = _sc_aggregate(h, colh, rowh, zeros2d, zeros1d, ones1,
                            n_pad, nch0, nch1, with_deg=False)
    logits = _tc_linear(agg2, degs3, W2, b2, relu=False)
    return logits[:N]


# speedup vs baseline: 4.5159x; 1.0013x over previous
import functools

import jax
import jax.numpy as jnp
from jax import lax
from jax.experimental import pallas as pl
from jax.experimental.pallas import tpu as pltpu
from jax.experimental.pallas import tpu_sc as plsc

NC = 2
NS = 16
CHUNK = 128
KB = 8


def _round_up(a, b):
    return (a + b - 1) // b * b


def _sc_aggregate(data, colh, rowh, zeros2d, zeros1d, ones1, n_pad,
                  nch0, nch1, with_deg):
    D = data.shape[1]
    assert nch0 > 0 and nch1 > 0
    rpt = n_pad // NS
    mesh = plsc.VectorSubcoreMesh(core_axis_name="c", subcore_axis_name="s")

    out_type = [jax.ShapeDtypeStruct((NC, n_pad, D), jnp.float32)]
    deg_scratch = []
    if with_deg:
        out_type.append(jax.ShapeDtypeStruct((NC, n_pad), jnp.float32))
        deg_scratch = [pltpu.VMEM_SHARED((n_pad,), jnp.float32)]

    @functools.partial(
        pl.kernel,
        out_type=tuple(out_type),
        mesh=mesh,
        scratch_types=[
            pltpu.VMEM((2, KB, CHUNK), jnp.int32),
            pltpu.VMEM((2, KB, CHUNK), jnp.int32),
            pltpu.VMEM((2, CHUNK, D), jnp.float32),
            pltpu.VMEM((CHUNK,), jnp.float32),
            pltpu.VMEM_SHARED((n_pad, D), jnp.float32),
            *deg_scratch,
            pltpu.SemaphoreType.DMA,
            pltpu.SemaphoreType.DMA,
            pltpu.SemaphoreType.DMA,
        ],
    )
    def k(data_hbm, col_hbm, row_hbm, z2_hbm, z1_hbm, ones_hbm, *rest):
        if with_deg:
            (agg_out, deg_out, col_w, row_w, gbuf, ones_v, sh_agg, sh_deg,
             gsem, isem, ssem) = rest
        else:
            (agg_out, col_w, row_w, gbuf, ones_v, sh_agg,
             gsem, isem, ssem) = rest
        c = lax.axis_index("c")
        s = lax.axis_index("s")
        rbase = s * rpt
        base_t = jnp.where(c == 0, s * nch0, NS * nch0 + s * nch1)
        with jax.named_scope("zero"):
            pltpu.async_copy(z2_hbm, sh_agg.at[pl.ds(rbase, rpt)], ssem)
            if with_deg:
                pltpu.async_copy(z1_hbm, sh_deg.at[pl.ds(rbase, rpt)], ssem)
                pltpu.async_copy(ones_hbm, ones_v, ssem)
            pltpu.async_copy(col_hbm.at[pl.ds(base_t, KB)], col_w.at[0], isem)
            pltpu.async_copy(row_hbm.at[pl.ds(base_t, KB)], row_w.at[0], isem)
            pltpu.make_async_copy(col_hbm.at[pl.ds(base_t, KB)], col_w.at[0],
                                  isem).wait()
            pltpu.async_copy(data_hbm.at[col_w.at[0, 0]], gbuf.at[0], gsem)
            pltpu.async_copy(data_hbm.at[col_w.at[0, 1]], gbuf.at[1], gsem)
            pltpu.make_async_copy(z2_hbm, sh_agg.at[pl.ds(rbase, rpt)],
                                  ssem).wait()
            if with_deg:
                pltpu.make_async_copy(z1_hbm, sh_deg.at[pl.ds(rbase, rpt)],
                                      ssem).wait()
                pltpu.make_async_copy(ones_hbm, ones_v, ssem).wait()
            plsc.subcore_barrier()

        def run(base, nch):
            nw = nch // KB

            def gath_start(slot, jj, b):
                pltpu.async_copy(data_hbm.at[col_w.at[slot, jj]], gbuf.at[b],
                                 gsem)

            def gath_wait(slot, jj, b):
                pltpu.make_async_copy(data_hbm.at[col_w.at[slot, jj]],
                                      gbuf.at[b], gsem).wait()

            def scat_start(slot, jj, b):
                pltpu.async_copy(gbuf.at[b], sh_agg.at[row_w.at[slot, jj]],
                                 ssem, add=True)

            def scat_wait(slot, jj, b):
                pltpu.make_async_copy(gbuf.at[b],
                                      sh_agg.at[row_w.at[slot, jj]],
                                      ssem).wait()

            @pl.loop(0, nw, step=2)
            def _(ww):
                for slot in range(2):
                    w = ww + slot
                    wb = base + w * KB
                    @pl.when(w > 0)
                    def _():
                        pltpu.make_async_copy(
                            col_hbm.at[pl.ds(wb, KB)], col_w.at[slot],
                            isem).wait()
                    pltpu.make_async_copy(
                        row_hbm.at[pl.ds(wb, KB)], row_w.at[slot], isem).wait()

                    @pl.when(w + 1 < nw)
                    def _():
                        nb = base + (w + 1) * KB
                        pltpu.async_copy(col_hbm.at[pl.ds(nb, KB)],
                                         col_w.at[1 - slot], isem)
                        pltpu.async_copy(row_hbm.at[pl.ds(nb, KB)],
                                         row_w.at[1 - slot], isem)

                    @pl.when(w > 0)
                    def _():
                        gath_start(slot, 0, 0)
                        gath_start(slot, 1, 1)

                    @pl.loop(0, KB, step=2)
                    def _(jj):
                        gath_wait(slot, jj, 0)
                        scat_start(slot, jj, 0)
                        if with_deg:
                            pltpu.sync_copy(ones_v,
                                            sh_deg.at[row_w.at[slot, jj]],
                                            add=True)
                        gath_wait(slot, jj + 1, 1)
                        scat_start(slot, jj + 1, 1)
                        if with_deg:
                            pltpu.sync_copy(ones_v,
                                            sh_deg.at[row_w.at[slot, jj + 1]],
                                            add=True)

                        @pl.when(jj + 2 < KB)
                        def _():
                            scat_wait(slot, jj, 0)
                            gath_start(slot, jj + 2, 0)
                            scat_wait(slot, jj + 1, 1)
                            gath_start(slot, jj + 3, 1)

                    scat_wait(slot, KB - 2, 0)
                    scat_wait(slot, KB - 1, 1)

        with jax.named_scope("edges"):
            @pl.when(c == 0)
            def _():
                run(s * nch0, nch0)

            @pl.when(c == 1)
            def _():
                run(NS * nch0 + s * nch1, nch1)

            plsc.subcore_barrier()

        with jax.named_scope("flush"):
            pltpu.sync_copy(sh_agg.at[pl.ds(rbase, rpt)],
                            agg_out.at[c, pl.ds(rbase, rpt)])
            if with_deg:
                pltpu.sync_copy(sh_deg.at[pl.ds(rbase, rpt)],
                                deg_out.at[c, pl.ds(rbase, rpt)])

    return k(data, colh, rowh, zeros2d, zeros1d, ones1)


def _tc_linear(parts, degs3, W, b, relu, n_out):
    D = parts.shape[2]
    H = W.shape[1]
    BLK = 400
    assert n_out % BLK == 0

    def body(p_ref, d_ref, w_ref, b_ref, o_ref):
        agg = p_ref[0] + p_ref[1]
        deg = d_ref[0] + d_ref[1]
        dinv = jnp.where(deg > 0, lax.rsqrt(jnp.maximum(deg, 1.0)), 0.0)
        out = jnp.dot(agg * dinv, w_ref[...],
                      preferred_element_type=jnp.float32,
                      precision=lax.Precision.HIGHEST)
        out = out + b_ref[...]
        if relu:
            out = jnp.maximum(out, 0.0)
        o_ref[...] = out

    return pl.pallas_call(
        body,
        grid=(n_out // BLK,),
        in_specs=[
            pl.BlockSpec((NC, BLK, D), lambda i: (0, i, 0)),
            pl.BlockSpec((NC, BLK, 1), lambda i: (0, i, 0)),
            pl.BlockSpec((D, H), lambda i: (0, 0)),
            pl.BlockSpec((1, H), lambda i: (0, 0)),
        ],
        out_specs=pl.BlockSpec((BLK, H), lambda i: (i, 0)),
        out_shape=jax.ShapeDtypeStruct((n_out, H), jnp.float32),
    )(parts, degs3, W, b.reshape(1, H))


def kernel(x, edge_index, W1, b1, W2, b2, size):
    N, D = x.shape
    E = edge_index.shape[1]
    H = W1.shape[1]

    tot = _round_up(E, NS * CHUNK * 4 * KB) // (NS * CHUNK)
    nch1 = tot // 2
    nch0 = tot - nch1
    e_pad = NS * tot * CHUNK
    n_pad = _round_up(N + 1, 1024)

    row = edge_index[0]
    col = edge_index[1]
    pad = e_pad - E
    arange_pad = jnp.arange(pad, dtype=jnp.int32)
    rowp = jnp.concatenate([row, N + arange_pad % (n_pad - N)])
    colp = jnp.concatenate([col, arange_pad % N])
    rowh = rowp.reshape(NS * tot, CHUNK)
    colh = colp.reshape(NS * tot, CHUNK)

    rpt = n_pad // NS
    zeros2d = jnp.zeros((rpt, D), jnp.float32)
    zeros1d = jnp.zeros((rpt,), jnp.float32)
    ones1 = jnp.ones((CHUNK,), jnp.float32)

    agg1, deg = _sc_aggregate(x, colh, rowh, zeros2d, zeros1d, ones1,
                              n_pad, nch0, nch1, with_deg=True)
    degs3 = deg.reshape(NC, n_pad, 1)
    h = _tc_linear(agg1, degs3, W1, b1, relu=True, n_out=N)
    (agg2,) = _sc_aggregate(h, colh, rowh, zeros2d, zeros1d, ones1,
                            n_pad, nch0, nch1, with_deg=False)
    return _tc_linear(agg2, degs3, W2, b2, relu=False, n_out=N)


# speedup vs baseline: 4.5588x; 1.0095x over previous
import functools

import jax
import jax.numpy as jnp
from jax import lax
from jax.experimental import pallas as pl
from jax.experimental.pallas import tpu as pltpu
from jax.experimental.pallas import tpu_sc as plsc

NC = 2
NS = 16
CHUNK = 128
KB = 8


def _round_up(a, b):
    return (a + b - 1) // b * b


def _sc_aggregate(data, colh, rowh, zeros2d, zeros1d, ones1, n_pad,
                  nch0, nch1, with_deg):
    D = data.shape[1]
    assert nch0 > 0 and nch1 > 0
    rpt = n_pad // NS
    mesh = plsc.VectorSubcoreMesh(core_axis_name="c", subcore_axis_name="s")

    out_type = [jax.ShapeDtypeStruct((NC, n_pad, D), jnp.float32)]
    deg_scratch = []
    if with_deg:
        out_type.append(jax.ShapeDtypeStruct((NC, n_pad), jnp.float32))
        deg_scratch = [pltpu.VMEM_SHARED((n_pad,), jnp.float32)]

    @functools.partial(
        pl.kernel,
        out_type=tuple(out_type),
        mesh=mesh,
        scratch_types=[
            pltpu.VMEM((2, KB, CHUNK), jnp.int32),
            pltpu.VMEM((2, KB, CHUNK), jnp.int32),
            pltpu.VMEM((2, CHUNK, D), jnp.float32),
            pltpu.VMEM((CHUNK,), jnp.float32),
            pltpu.VMEM_SHARED((n_pad, D), jnp.float32),
            *deg_scratch,
            pltpu.SemaphoreType.DMA,
            pltpu.SemaphoreType.DMA,
            pltpu.SemaphoreType.DMA,
        ],
    )
    def k(data_hbm, col_hbm, row_hbm, z2_hbm, z1_hbm, ones_hbm, *rest):
        if with_deg:
            (agg_out, deg_out, col_w, row_w, gbuf, ones_v, sh_agg, sh_deg,
             gsem, isem, ssem) = rest
        else:
            (agg_out, col_w, row_w, gbuf, ones_v, sh_agg,
             gsem, isem, ssem) = rest
        c = lax.axis_index("c")
        s = lax.axis_index("s")
        rbase = s * rpt
        base_t = jnp.where(c == 0, s * nch0, NS * nch0 + s * nch1)
        with jax.named_scope("zero"):
            pltpu.async_copy(z2_hbm, sh_agg.at[pl.ds(rbase, rpt)], ssem)
            if with_deg:
                pltpu.async_copy(z1_hbm, sh_deg.at[pl.ds(rbase, rpt)], ssem)
                pltpu.async_copy(ones_hbm, ones_v, ssem)
            pltpu.async_copy(col_hbm.at[pl.ds(base_t, KB)], col_w.at[0], isem)
            pltpu.async_copy(row_hbm.at[pl.ds(base_t, KB)], row_w.at[0], isem)
            pltpu.make_async_copy(col_hbm.at[pl.ds(base_t, KB)], col_w.at[0],
                                  isem).wait()
            pltpu.async_copy(data_hbm.at[col_w.at[0, 0]], gbuf.at[0], gsem)
            pltpu.async_copy(data_hbm.at[col_w.at[0, 1]], gbuf.at[1], gsem)
            pltpu.make_async_copy(z2_hbm, sh_agg.at[pl.ds(rbase, rpt)],
                                  ssem).wait()
            if with_deg:
                pltpu.make_async_copy(z1_hbm, sh_deg.at[pl.ds(rbase, rpt)],
                                      ssem).wait()
                pltpu.make_async_copy(ones_hbm, ones_v, ssem).wait()
            plsc.subcore_barrier()

        def run(base, nch):
            nw = nch // KB

            def gath_start(slot, jj, b):
                pltpu.async_copy(data_hbm.at[col_w.at[slot, jj]], gbuf.at[b],
                                 gsem)

            def gath_wait(slot, jj, b):
                pltpu.make_async_copy(data_hbm.at[col_w.at[slot, jj]],
                                      gbuf.at[b], gsem).wait()

            def scat_start(slot, jj, b):
                pltpu.async_copy(gbuf.at[b], sh_agg.at[row_w.at[slot, jj]],
                                 ssem, add=True)

            def scat_wait(slot, jj, b):
                pltpu.make_async_copy(gbuf.at[b],
                                      sh_agg.at[row_w.at[slot, jj]],
                                      ssem).wait()

            @pl.loop(0, nw, step=2)
            def _(ww):
                for slot in range(2):
                    w = ww + slot
                    wb = base + w * KB
                    @pl.when(w > 0)
                    def _():
                        pltpu.make_async_copy(
                            col_hbm.at[pl.ds(wb, KB)], col_w.at[slot],
                            isem).wait()
                    pltpu.make_async_copy(
                        row_hbm.at[pl.ds(wb, KB)], row_w.at[slot], isem).wait()

                    @pl.when(w + 1 < nw)
                    def _():
                        nb = base + (w + 1) * KB
                        pltpu.async_copy(col_hbm.at[pl.ds(nb, KB)],
                                         col_w.at[1 - slot], isem)
                        pltpu.async_copy(row_hbm.at[pl.ds(nb, KB)],
                                         row_w.at[1 - slot], isem)

                    @pl.when(w > 0)
                    def _():
                        gath_start(slot, 0, 0)
                        gath_start(slot, 1, 1)

                    for jj in range(0, KB, 2):
                        gath_wait(slot, jj, 0)
                        scat_start(slot, jj, 0)
                        if with_deg:
                            pltpu.sync_copy(ones_v,
                                            sh_deg.at[row_w.at[slot, jj]],
                                            add=True)
                        gath_wait(slot, jj + 1, 1)
                        scat_start(slot, jj + 1, 1)
                        if with_deg:
                            pltpu.sync_copy(ones_v,
                                            sh_deg.at[row_w.at[slot, jj + 1]],
                                            add=True)
                        if jj + 2 < KB:
                            scat_wait(slot, jj, 0)
                            gath_start(slot, jj + 2, 0)
                            scat_wait(slot, jj + 1, 1)
                            gath_start(slot, jj + 3, 1)

                    scat_wait(slot, KB - 2, 0)
                    scat_wait(slot, KB - 1, 1)

        with jax.named_scope("edges"):
            @pl.when(c == 0)
            def _():
                run(s * nch0, nch0)

            @pl.when(c == 1)
            def _():
                run(NS * nch0 + s * nch1, nch1)

            plsc.subcore_barrier()

        with jax.named_scope("flush"):
            pltpu.sync_copy(sh_agg.at[pl.ds(rbase, rpt)],
                            agg_out.at[c, pl.ds(rbase, rpt)])
            if with_deg:
                pltpu.sync_copy(sh_deg.at[pl.ds(rbase, rpt)],
                                deg_out.at[c, pl.ds(rbase, rpt)])

    return k(data, colh, rowh, zeros2d, zeros1d, ones1)


def _tc_linear(parts, degs3, W, b, relu, n_out):
    D = parts.shape[2]
    H = W.shape[1]
    BLK = 512 if n_out % 512 == 0 else 400
    assert n_out % BLK == 0

    def body(p_ref, d_ref, w_ref, b_ref, o_ref):
        agg = p_ref[0] + p_ref[1]
        deg = d_ref[0] + d_ref[1]
        dinv = jnp.where(deg > 0, lax.rsqrt(jnp.maximum(deg, 1.0)), 0.0)
        out = jnp.dot(agg * dinv, w_ref[...],
                      preferred_element_type=jnp.float32,
                      precision=lax.Precision.HIGHEST)
        out = out + b_ref[...]
        if relu:
            out = jnp.maximum(out, 0.0)
        o_ref[...] = out

    return pl.pallas_call(
        body,
        grid=(n_out // BLK,),
        in_specs=[
            pl.BlockSpec((NC, BLK, D), lambda i: (0, i, 0)),
            pl.BlockSpec((NC, BLK, 1), lambda i: (0, i, 0)),
            pl.BlockSpec((D, H), lambda i: (0, 0)),
            pl.BlockSpec((1, H), lambda i: (0, 0)),
        ],
        out_specs=pl.BlockSpec((BLK, H), lambda i: (i, 0)),
        out_shape=jax.ShapeDtypeStruct((n_out, H), jnp.float32),
    )(parts, degs3, W, b.reshape(1, H))


def kernel(x, edge_index, W1, b1, W2, b2, size):
    N, D = x.shape
    E = edge_index.shape[1]
    H = W1.shape[1]

    tot = _round_up(E, NS * CHUNK * 4 * KB) // (NS * CHUNK)
    nch1 = tot // 2
    nch0 = tot - nch1
    e_pad = NS * tot * CHUNK
    n_pad = _round_up(N + 1, 1024)

    row = edge_index[0]
    col = edge_index[1]
    pad = e_pad - E
    arange_pad = jnp.arange(pad, dtype=jnp.int32)
    rowp = jnp.concatenate([row, N + arange_pad % (n_pad - N)])
    colp = jnp.concatenate([col, arange_pad % N])
    rowh = rowp.reshape(NS * tot, CHUNK)
    colh = colp.reshape(NS * tot, CHUNK)

    rpt = n_pad // NS
    zeros2d = jnp.zeros((rpt, D), jnp.float32)
    zeros1d = jnp.zeros((rpt,), jnp.float32)
    ones1 = jnp.ones((CHUNK,), jnp.float32)

    agg1, deg = _sc_aggregate(x, colh, rowh, zeros2d, zeros1d, ones1,
                              n_pad, nch0, nch1, with_deg=True)
    degs3 = deg.reshape(NC, n_pad, 1)
    h = _tc_linear(agg1, degs3, W1, b1, relu=True, n_out=n_pad)
    (agg2,) = _sc_aggregate(h, colh, rowh, zeros2d, zeros1d, ones1,
                            n_pad, nch0, nch1, with_deg=False)
    return _tc_linear(agg2, degs3, W2, b2, relu=False, n_out=N)


# speedup vs baseline: 4.6295x; 1.0155x over previous
import functools

import jax
import jax.numpy as jnp
from jax import lax
from jax.experimental import pallas as pl
from jax.experimental.pallas import tpu as pltpu
from jax.experimental.pallas import tpu_sc as plsc

NC = 2
NS = 16
CHUNK = 128
KB = 8


def _round_up(a, b):
    return (a + b - 1) // b * b


def _sc_aggregate(data, colh, rowh, zeros2d, zeros1d, ones1, n_pad,
                  nch0, nch1, with_deg):
    D = data.shape[1]
    assert nch0 > 0 and nch1 > 0
    rpt = n_pad // NS
    mesh = plsc.VectorSubcoreMesh(core_axis_name="c", subcore_axis_name="s")

    out_type = [jax.ShapeDtypeStruct((NC, n_pad, D), jnp.float32)]
    deg_scratch = []
    if with_deg:
        out_type.append(jax.ShapeDtypeStruct((NC, n_pad), jnp.float32))
        deg_scratch = [pltpu.VMEM_SHARED((n_pad,), jnp.float32)]

    @functools.partial(
        pl.kernel,
        out_type=tuple(out_type),
        mesh=mesh,
        scratch_types=[
            pltpu.VMEM((2, KB, CHUNK), jnp.int32),
            pltpu.VMEM((2, KB, CHUNK), jnp.int32),
            pltpu.VMEM((2, CHUNK, D), jnp.float32),
            pltpu.VMEM((CHUNK,), jnp.float32),
            pltpu.VMEM_SHARED((n_pad, D), jnp.float32),
            *deg_scratch,
            pltpu.SemaphoreType.DMA,
            pltpu.SemaphoreType.DMA,
            pltpu.SemaphoreType.DMA,
        ],
    )
    def k(data_hbm, col_hbm, row_hbm, z2_hbm, z1_hbm, ones_hbm, *rest):
        if with_deg:
            (agg_out, deg_out, col_w, row_w, gbuf, ones_v, sh_agg, sh_deg,
             gsem, isem, ssem) = rest
        else:
            (agg_out, col_w, row_w, gbuf, ones_v, sh_agg,
             gsem, isem, ssem) = rest
        c = lax.axis_index("c")
        s = lax.axis_index("s")
        rbase = s * rpt
        base_t = jnp.where(c == 0, s * nch0, NS * nch0 + s * nch1)
        with jax.named_scope("zero"):
            pltpu.async_copy(z2_hbm, sh_agg.at[pl.ds(rbase, rpt)], ssem)
            if with_deg:
                pltpu.async_copy(z1_hbm, sh_deg.at[pl.ds(rbase, rpt)], ssem)
                pltpu.async_copy(ones_hbm, ones_v, ssem)
            pltpu.async_copy(col_hbm.at[pl.ds(base_t, KB)], col_w.at[0], isem)
            pltpu.async_copy(row_hbm.at[pl.ds(base_t, KB)], row_w.at[0], isem)
            pltpu.make_async_copy(col_hbm.at[pl.ds(base_t, KB)], col_w.at[0],
                                  isem).wait()
            pltpu.async_copy(data_hbm.at[col_w.at[0, 0]], gbuf.at[0], gsem)
            pltpu.async_copy(data_hbm.at[col_w.at[0, 1]], gbuf.at[1], gsem)
            pltpu.make_async_copy(z2_hbm, sh_agg.at[pl.ds(rbase, rpt)],
                                  ssem).wait()
            if with_deg:
                pltpu.make_async_copy(z1_hbm, sh_deg.at[pl.ds(rbase, rpt)],
                                      ssem).wait()
                pltpu.make_async_copy(ones_hbm, ones_v, ssem).wait()
            plsc.subcore_barrier()

        def run(base, nch):
            nw = nch // KB

            def gath_start(slot, jj, b):
                pltpu.async_copy(data_hbm.at[col_w.at[slot, jj]], gbuf.at[b],
                                 gsem)

            def gath_wait(slot, jj, b):
                pltpu.make_async_copy(data_hbm.at[col_w.at[slot, jj]],
                                      gbuf.at[b], gsem).wait()

            def scat_start(slot, jj, b):
                pltpu.async_copy(gbuf.at[b], sh_agg.at[row_w.at[slot, jj]],
                                 ssem, add=True)

            def scat_wait(slot, jj, b):
                pltpu.make_async_copy(gbuf.at[b],
                                      sh_agg.at[row_w.at[slot, jj]],
                                      ssem).wait()

            @pl.loop(0, nw, step=2)
            def _(ww):
                for slot in range(2):
                    w = ww + slot
                    wb = base + w * KB
                    @pl.when(w > 0)
                    def _():
                        pltpu.make_async_copy(
                            col_hbm.at[pl.ds(wb, KB)], col_w.at[slot],
                            isem).wait()
                    pltpu.make_async_copy(
                        row_hbm.at[pl.ds(wb, KB)], row_w.at[slot], isem).wait()

                    @pl.when(w + 1 < nw)
                    def _():
                        nb = base + (w + 1) * KB
                        pltpu.async_copy(col_hbm.at[pl.ds(nb, KB)],
                                         col_w.at[1 - slot], isem)
                        pltpu.async_copy(row_hbm.at[pl.ds(nb, KB)],
                                         row_w.at[1 - slot], isem)

                    @pl.when(w > 0)
                    def _():
                        gath_start(slot, 0, 0)
                        gath_start(slot, 1, 1)

                    for jj in range(0, KB, 2):
                        gath_wait(slot, jj, 0)
                        scat_start(slot, jj, 0)
                        if with_deg:
                            pltpu.sync_copy(ones_v,
                                            sh_deg.at[row_w.at[slot, jj]],
                                            add=True)
                        gath_wait(slot, jj + 1, 1)
                        scat_start(slot, jj + 1, 1)
                        if with_deg:
                            pltpu.sync_copy(ones_v,
                                            sh_deg.at[row_w.at[slot, jj + 1]],
                                            add=True)
                        if jj + 2 < KB:
                            scat_wait(slot, jj, 0)
                            gath_start(slot, jj + 2, 0)
                            scat_wait(slot, jj + 1, 1)
                            gath_start(slot, jj + 3, 1)

                    scat_wait(slot, KB - 2, 0)
                    scat_wait(slot, KB - 1, 1)

        with jax.named_scope("edges"):
            @pl.when(c == 0)
            def _():
                run(s * nch0, nch0)

            @pl.when(c == 1)
            def _():
                run(NS * nch0 + s * nch1, nch1)

            plsc.subcore_barrier()

        with jax.named_scope("flush"):
            pltpu.sync_copy(sh_agg.at[pl.ds(rbase, rpt)],
                            agg_out.at[c, pl.ds(rbase, rpt)])
            if with_deg:
                pltpu.sync_copy(sh_deg.at[pl.ds(rbase, rpt)],
                                deg_out.at[c, pl.ds(rbase, rpt)])

    return k(data, colh, rowh, zeros2d, zeros1d, ones1)


def _prep_indices(eir, E, tot_rows, N, n_pad):
    BLKR = 320
    assert tot_rows % BLKR == 0
    nreal = E // CHUNK

    def body(ei_ref, rh_ref, ch_ref):
        i = pl.program_id(0)
        rid = lax.broadcasted_iota(jnp.int32, (BLKR, CHUNK), 0) + i * BLKR
        cid = lax.broadcasted_iota(jnp.int32, (BLKR, CHUNK), 1)
        k = (rid - nreal) * CHUNK + cid
        real = rid < nreal
        rh_ref[...] = jnp.where(real, ei_ref[0], N + k % (n_pad - N))
        ch_ref[...] = jnp.where(real, ei_ref[1], k % N)

    rowh, colh = pl.pallas_call(
        body,
        grid=(tot_rows // BLKR,),
        in_specs=[pl.BlockSpec((2, BLKR, CHUNK), lambda i: (0, i, 0))],
        out_specs=[pl.BlockSpec((BLKR, CHUNK), lambda i: (i, 0)),
                   pl.BlockSpec((BLKR, CHUNK), lambda i: (i, 0))],
        out_shape=[jax.ShapeDtypeStruct((tot_rows, CHUNK), jnp.int32),
                   jax.ShapeDtypeStruct((tot_rows, CHUNK), jnp.int32)],
    )(eir)
    return rowh, colh


def _tc_linear(parts, degs3, W, b, relu, n_out):
    D = parts.shape[2]
    H = W.shape[1]
    BLK = 512 if n_out % 512 == 0 else 400
    assert n_out % BLK == 0

    def body(p_ref, d_ref, w_ref, b_ref, o_ref):
        agg = p_ref[0] + p_ref[1]
        deg = d_ref[0] + d_ref[1]
        dinv = jnp.where(deg > 0, lax.rsqrt(jnp.maximum(deg, 1.0)), 0.0)
        out = jnp.dot(agg * dinv, w_ref[...],
                      preferred_element_type=jnp.float32,
                      precision=lax.Precision.HIGHEST)
        out = out + b_ref[...]
        if relu:
            out = jnp.maximum(out, 0.0)
        o_ref[...] = out

    return pl.pallas_call(
        body,
        grid=(n_out // BLK,),
        in_specs=[
            pl.BlockSpec((NC, BLK, D), lambda i: (0, i, 0)),
            pl.BlockSpec((NC, BLK, 1), lambda i: (0, i, 0)),
            pl.BlockSpec((D, H), lambda i: (0, 0)),
            pl.BlockSpec((1, H), lambda i: (0, 0)),
        ],
        out_specs=pl.BlockSpec((BLK, H), lambda i: (i, 0)),
        out_shape=jax.ShapeDtypeStruct((n_out, H), jnp.float32),
    )(parts, degs3, W, b.reshape(1, H))


def kernel(x, edge_index, W1, b1, W2, b2, size):
    N, D = x.shape
    E = edge_index.shape[1]
    H = W1.shape[1]

    tot = _round_up(E, NS * CHUNK * 4 * KB) // (NS * CHUNK)
    nch1 = tot // 2
    nch0 = tot - nch1
    e_pad = NS * tot * CHUNK
    n_pad = _round_up(N + 1, 1024)

    if E % CHUNK == 0:
        rowh, colh = _prep_indices(edge_index.reshape(2, E // CHUNK, CHUNK),
                                   E, NS * tot, N, n_pad)
    else:
        pad = e_pad - E
        arange_pad = jnp.arange(pad, dtype=jnp.int32)
        rowp = jnp.concatenate([edge_index[0], N + arange_pad % (n_pad - N)])
        colp = jnp.concatenate([edge_index[1], arange_pad % N])
        rowh = rowp.reshape(NS * tot, CHUNK)
        colh = colp.reshape(NS * tot, CHUNK)

    rpt = n_pad // NS
    zeros2d = jnp.zeros((rpt, D), jnp.float32)
    zeros1d = jnp.zeros((rpt,), jnp.float32)
    ones1 = jnp.ones((CHUNK,), jnp.float32)

    agg1, deg = _sc_aggregate(x, colh, rowh, zeros2d, zeros1d, ones1,
                              n_pad, nch0, nch1, with_deg=True)
    degs3 = deg.reshape(NC, n_pad, 1)
    h = _tc_linear(agg1, degs3, W1, b1, relu=True, n_out=n_pad)
    (agg2,) = _sc_aggregate(h, colh, rowh, zeros2d, zeros1d, ones1,
                            n_pad, nch0, nch1, with_deg=False)
    return _tc_linear(agg2, degs3, W2, b2, relu=False, n_out=N)
